# baseline (speedup 1.0000x reference)
import jax
import jax.numpy as jnp
from jax.experimental import pallas as pl
from jax.experimental.pallas import tpu as pltpu

_EPS = 1e-12


def _make_kernel(B, N, M, D, TN):
    NI = N // TN

    CH = 128
    NCH = TN // CH

    def _chamfer_kernel(x_ref, t_ref, out_ref, colacc_ref, t2s_ref):
        b = pl.program_id(0)
        i = pl.program_id(1)
        xb = x_ref[0]

        @pl.when(i == 0)
        def _():
            tb0 = t_ref[0]
            t2s_ref[...] = jnp.sum(tb0 * tb0, axis=0, keepdims=True)

        tb = t_ref[0]
        t2s = t2s_ref[...]

        row_contrib = 0.0
        colparts = []
        for c in range(NCH):
            xc = xb[c * CH:(c + 1) * CH]
            cross2 = jax.lax.dot_general(
                xc + xc, tb, (((1,), (0,)), ((), ())),
                precision=jax.lax.Precision.DEFAULT,
                preferred_element_type=jnp.float32,
            )
            x2c = jnp.sum(xc * xc, axis=1, keepdims=True)
            rowpart = jnp.min(t2s - cross2, axis=1, keepdims=True)
            rowmin = x2c + rowpart
            row_contrib += jnp.sum(jnp.sqrt(jnp.maximum(rowmin, _EPS)))
            colparts.append(jnp.min(x2c - cross2, axis=0, keepdims=True))
        row_contrib = row_contrib / (N * B)
        colpart = colparts[0]
        for cp in colparts[1:]:
            colpart = jnp.minimum(colpart, cp)

        @pl.when(jnp.logical_and(b == 0, i == 0))
        def _():
            out_ref[...] = jnp.zeros_like(out_ref)

        @pl.when(i == 0)
        def _():
            colacc_ref[...] = colpart

        @pl.when(i > 0)
        def _():
            colacc_ref[...] = jnp.minimum(colacc_ref[...], colpart)

        out_ref[...] += row_contrib

        @pl.when(i == NI - 1)
        def _():
            colmin = t2s_ref[...] + colacc_ref[...]
            col_sqrt = jnp.sqrt(jnp.maximum(colmin, _EPS))
            out_ref[...] += jnp.sum(col_sqrt) / (M * B)

    return _chamfer_kernel, NI


def _chamfer(x, tt, interpret=False):
    B, N, D = x.shape
    M = tt.shape[2]
    TN = 512
    kern, NI = _make_kernel(B, N, M, D, TN)
    out = pl.pallas_call(
        kern,
        grid=(B, NI),
        in_specs=[
            pl.BlockSpec((1, TN, D), lambda b, i: (b, i, 0)),
            pl.BlockSpec((1, D, M), lambda b, i: (b, 0, 0)),
        ],
        out_specs=pl.BlockSpec((1, 1), lambda b, i: (0, 0)),
        out_shape=jax.ShapeDtypeStruct((1, 1), jnp.float32),
        scratch_shapes=[
            pltpu.VMEM((1, M), jnp.float32),
            pltpu.VMEM((1, M), jnp.float32),
        ],
        interpret=interpret,
    )(x, tt)
    return out[0, 0]


@jax.jit
def _chamfer_jit(x, tt):
    return _chamfer(x, tt)


def kernel(x, target):
    tt = jnp.swapaxes(target, 1, 2)
    return _chamfer_jit(x, tt)


# speedup vs baseline: 1.0215x; 1.0215x over previous
import jax
import jax.numpy as jnp
from jax.experimental import pallas as pl
from jax.experimental.pallas import tpu as pltpu

_EPS = 1e-12


def _make_kernel(B, N, M, D, TN):
    NI = N // TN

    CH = 256
    NCH = TN // CH

    def _chamfer_kernel(x_ref, t_ref, out_ref, colacc_ref, t2s_ref):
        b = pl.program_id(0)
        i = pl.program_id(1)
        xb = x_ref[0]

        @pl.when(i == 0)
        def _():
            tb0 = t_ref[0]
            t2s_ref[...] = jnp.sum(tb0 * tb0, axis=0, keepdims=True)

        tb = t_ref[0]
        t2s = t2s_ref[...]

        row_contrib = 0.0
        colparts = []
        for c in range(NCH):
            xc = xb[c * CH:(c + 1) * CH]
            cross2 = jax.lax.dot_general(
                xc + xc, tb, (((1,), (0,)), ((), ())),
                precision=jax.lax.Precision.DEFAULT,
                preferred_element_type=jnp.float32,
            )
            x2c = jnp.sum(xc * xc, axis=1, keepdims=True)
            rowpart = jnp.min(t2s - cross2, axis=1, keepdims=True)
            rowmin = x2c + rowpart
            row_contrib += jnp.sum(jnp.sqrt(jnp.maximum(rowmin, _EPS)))
            colparts.append(jnp.min(x2c - cross2, axis=0, keepdims=True))
        row_contrib = row_contrib / (N * B)
        colpart = colparts[0]
        for cp in colparts[1:]:
            colpart = jnp.minimum(colpart, cp)

        @pl.when(jnp.logical_and(b == 0, i == 0))
        def _():
            out_ref[...] = jnp.zeros_like(out_ref)

        @pl.when(i == 0)
        def _():
            colacc_ref[...] = colpart

        @pl.when(i > 0)
        def _():
            colacc_ref[...] = jnp.minimum(colacc_ref[...], colpart)

        out_ref[...] += row_contrib

        @pl.when(i == NI - 1)
        def _():
            colmin = t2s_ref[...] + colacc_ref[...]
            col_sqrt = jnp.sqrt(jnp.maximum(colmin, _EPS))
            out_ref[...] += jnp.sum(col_sqrt) / (M * B)

    return _chamfer_kernel, NI


def _chamfer(x, tt, interpret=False):
    B, N, D = x.shape
    M = tt.shape[2]
    TN = 512
    kern, NI = _make_kernel(B, N, M, D, TN)
    out = pl.pallas_call(
        kern,
        grid=(B, NI),
        in_specs=[
            pl.BlockSpec((1, TN, D), lambda b, i: (b, i, 0)),
            pl.BlockSpec((1, D, M), lambda b, i: (b, 0, 0)),
        ],
        out_specs=pl.BlockSpec((1, 1), lambda b, i: (0, 0)),
        out_shape=jax.ShapeDtypeStruct((1, 1), jnp.float32),
        scratch_shapes=[
            pltpu.VMEM((1, M), jnp.float32),
            pltpu.VMEM((1, M), jnp.float32),
        ],
        interpret=interpret,
    )(x, tt)
    return out[0, 0]


@jax.jit
def _chamfer_jit(x, tt):
    return _chamfer(x, tt)


def kernel(x, target):
    tt = jnp.swapaxes(target, 1, 2)
    return _chamfer_jit(x, tt)


# speedup vs baseline: 1.0695x; 1.0470x over previous
import jax
import jax.numpy as jnp
from jax.experimental import pallas as pl
from jax.experimental.pallas import tpu as pltpu

_EPS = 1e-12


def _make_kernel(B, N, M, D, TN):
    NI = N // TN

    MC = 2048
    NMC = M // MC

    def _chamfer_kernel(x_ref, t_ref, out_ref, colacc_ref, t2s_ref):
        b = pl.program_id(0)
        i = pl.program_id(1)
        xb = x_ref[0]

        @pl.when(i == 0)
        def _():
            tb0 = t_ref[0]
            t2s_ref[...] = jnp.sum(tb0 * tb0, axis=0, keepdims=True)

        tb = t_ref[0]
        xb2 = xb + xb
        x2s = jnp.sum(xb * xb, axis=1, keepdims=True)

        rowpart = None
        colparts = []
        for c in range(NMC):
            lo, hi = c * MC, (c + 1) * MC
            cross2 = jax.lax.dot_general(
                xb2, tb[:, lo:hi], (((1,), (0,)), ((), ())),
                precision=jax.lax.Precision.DEFAULT,
                preferred_element_type=jnp.float32,
            )
            t2c = t2s_ref[0:1, lo:hi]
            rp = jnp.min(t2c - cross2, axis=1, keepdims=True)
            rowpart = rp if rowpart is None else jnp.minimum(rowpart, rp)
            colparts.append(jnp.min(x2s - cross2, axis=0, keepdims=True))
        rowmin = x2s + rowpart
        row_contrib = jnp.sum(jnp.sqrt(jnp.maximum(rowmin, _EPS))) / (N * B)
        colpart = jnp.concatenate(colparts, axis=1)

        @pl.when(jnp.logical_and(b == 0, i == 0))
        def _():
            out_ref[...] = jnp.zeros_like(out_ref)

        @pl.when(i == 0)
        def _():
            colacc_ref[...] = colpart

        @pl.when(i > 0)
        def _():
            colacc_ref[...] = jnp.minimum(colacc_ref[...], colpart)

        out_ref[...] += row_contrib

        @pl.when(i == NI - 1)
        def _():
            colmin = t2s_ref[...] + colacc_ref[...]
            col_sqrt = jnp.sqrt(jnp.maximum(colmin, _EPS))
            out_ref[...] += jnp.sum(col_sqrt) / (M * B)

    return _chamfer_kernel, NI


def _chamfer(x, tt, interpret=False):
    B, N, D = x.shape
    M = tt.shape[2]
    TN = 512
    kern, NI = _make_kernel(B, N, M, D, TN)
    out = pl.pallas_call(
        kern,
        grid=(B, NI),
        in_specs=[
            pl.BlockSpec((1, TN, D), lambda b, i: (b, i, 0)),
            pl.BlockSpec((1, D, M), lambda b, i: (b, 0, 0)),
        ],
        out_specs=pl.BlockSpec((1, 1), lambda b, i: (0, 0)),
        out_shape=jax.ShapeDtypeStruct((1, 1), jnp.float32),
        scratch_shapes=[
            pltpu.VMEM((1, M), jnp.float32),
            pltpu.VMEM((1, M), jnp.float32),
        ],
        interpret=interpret,
    )(x, tt)
    return out[0, 0]


@jax.jit
def _chamfer_jit(x, tt):
    return _chamfer(x, tt)


def kernel(x, target):
    tt = jnp.swapaxes(target, 1, 2)
    return _chamfer_jit(x, tt)


# speedup vs baseline: 1.0888x; 1.0181x over previous
import jax
import jax.numpy as jnp
from jax.experimental import pallas as pl
from jax.experimental.pallas import tpu as pltpu

_EPS = 1e-12


def _make_kernel(B, N, M, D, TN):
    NI = N // TN

    MC = M
    NMC = M // MC

    def _chamfer_kernel(x_ref, t_ref, out_ref, colacc_ref, t2s_ref):
        b = pl.program_id(0)
        i = pl.program_id(1)
        xb = x_ref[0]

        @pl.when(i == 0)
        def _():
            tb0 = t_ref[0]
            t2s_ref[...] = jnp.sum(tb0 * tb0, axis=0, keepdims=True)

        tb = t_ref[0]
        xb2 = xb + xb
        x2s = jnp.sum(xb * xb, axis=1, keepdims=True)

        rowpart = None
        colparts = []
        for c in range(NMC):
            lo, hi = c * MC, (c + 1) * MC
            cross2 = jax.lax.dot_general(
                xb2, tb[:, lo:hi], (((1,), (0,)), ((), ())),
                precision=jax.lax.Precision.DEFAULT,
                preferred_element_type=jnp.float32,
            )
            t2c = t2s_ref[0:1, lo:hi]
            rp = jnp.min(t2c - cross2, axis=1, keepdims=True)
            rowpart = rp if rowpart is None else jnp.minimum(rowpart, rp)
            colparts.append(jnp.min(x2s - cross2, axis=0, keepdims=True))
        rowmin = x2s + rowpart
        row_contrib = jnp.sum(jnp.sqrt(jnp.maximum(rowmin, _EPS))) / (N * B)
        colpart = jnp.concatenate(colparts, axis=1)

        @pl.when(jnp.logical_and(b == 0, i == 0))
        def _():
            out_ref[...] = jnp.zeros_like(out_ref)

        @pl.when(i == 0)
        def _():
            colacc_ref[...] = colpart

        @pl.when(i > 0)
        def _():
            colacc_ref[...] = jnp.minimum(colacc_ref[...], colpart)

        out_ref[...] += row_contrib

        @pl.when(i == NI - 1)
        def _():
            colmin = t2s_ref[...] + colacc_ref[...]
            col_sqrt = jnp.sqrt(jnp.maximum(colmin, _EPS))
            out_ref[...] += jnp.sum(col_sqrt) / (M * B)

    return _chamfer_kernel, NI


def _chamfer(x, tt, interpret=False):
    B, N, D = x.shape
    M = tt.shape[2]
    TN = 512
    kern, NI = _make_kernel(B, N, M, D, TN)
    out = pl.pallas_call(
        kern,
        grid=(B, NI),
        in_specs=[
            pl.BlockSpec((1, TN, D), lambda b, i: (b, i, 0)),
            pl.BlockSpec((1, D, M), lambda b, i: (b, 0, 0)),
        ],
        out_specs=pl.BlockSpec((1, 1), lambda b, i: (0, 0)),
        out_shape=jax.ShapeDtypeStruct((1, 1), jnp.float32),
        scratch_shapes=[
            pltpu.VMEM((1, M), jnp.float32),
            pltpu.VMEM((1, M), jnp.float32),
        ],
        interpret=interpret,
    )(x, tt)
    return out[0, 0]


@jax.jit
def _chamfer_jit(x, tt):
    return _chamfer(x, tt)


def kernel(x, target):
    tt = jnp.swapaxes(target, 1, 2)
    return _chamfer_jit(x, tt)
